# Initial kernel scaffold; baseline (speedup 1.0000x reference)
import functools

import jax
import jax.numpy as jnp
from jax import lax
from jax.experimental import pallas as pl
from jax.experimental.pallas import tpu as pltpu
from jax.experimental.pallas import tpu_sc as plsc

N = 10000
E = 320000
D = 128
H = 64

NC = 2
NS = 16
NW = NC * NS

ROWS = E // 128
ROWS_PAD = 2560
ROWS_PER_W = ROWS_PAD // NW
E_PAD = ROWS_PAD * 128
GR = 2
GROUPS = ROWS_PER_W // GR

BE = 2560
NBLK = E // BE
QW = BE // 128

_mesh = plsc.VectorSubcoreMesh(core_axis_name="c", subcore_axis_name="s",
                               num_cores=NC, num_subcores=NS)


def _sc_gather_body(x_hbm, src_hbm, dst_hbm, xi_hbm, xj_hbm,
                    isrc_v, idst_v, xi_v, xj_v, sem):
    c = lax.axis_index("c")
    s = lax.axis_index("s")
    wid = s * NC + c
    row0 = wid * ROWS_PER_W

    @pl.loop(0, GROUPS)
    def _group(t):
        r0 = row0 + t * GR
        pltpu.sync_copy(src_hbm.at[pl.ds(r0, GR)], isrc_v)
        pltpu.sync_copy(dst_hbm.at[pl.ds(r0, GR)], idst_v)
        cps = []
        for j in range(GR):
            cps.append(pltpu.async_copy(
                x_hbm.at[isrc_v.at[j]], xj_v.at[pl.ds(j * 128, 128)], sem))
            cps.append(pltpu.async_copy(
                x_hbm.at[idst_v.at[j]], xi_v.at[pl.ds(j * 128, 128)], sem))
        for cp in cps:
            cp.wait()
        pltpu.sync_copy(xj_v, xj_hbm.at[pl.ds(r0 * 128, GR * 128)])
        pltpu.sync_copy(xi_v, xi_hbm.at[pl.ds(r0 * 128, GR * 128)])


_sc_gather = functools.partial(
    pl.kernel,
    out_type=[jax.ShapeDtypeStruct((E_PAD, D), jnp.float32),
              jax.ShapeDtypeStruct((E_PAD, D), jnp.float32)],
    mesh=_mesh,
    scratch_types=[
        pltpu.VMEM((GR, 128), jnp.int32),
        pltpu.VMEM((GR, 128), jnp.int32),
        pltpu.VMEM((GR * 128, D), jnp.float32),
        pltpu.VMEM((GR * 128, D), jnp.float32),
        pltpu.SemaphoreType.DMA,
    ],
)(_sc_gather_body)


def _sc_scatter_body(msg_hbm, dsti_hbm, zeros_hbm, out_hbm,
                     idx_v, msg_v, acc_sh):
    c = lax.axis_index("c")
    s = lax.axis_index("s")
    wid = s * NC + c

    @pl.when(s == 0)
    def _zero():
        pltpu.sync_copy(zeros_hbm, acc_sh)

    plsc.subcore_barrier()

    niter = (ROWS + NW - 1) // NW

    @pl.loop(0, niter)
    def _it(t):
        r = wid + NW * t

        @pl.when(r < ROWS)
        def _do():
            pltpu.sync_copy(dsti_hbm.at[r], idx_v)
            pltpu.sync_copy(msg_hbm.at[pl.ds(r * 128, 128)], msg_v)
            pltpu.sync_copy(msg_v, acc_sh.at[idx_v], add=True)

    plsc.subcore_barrier()

    @pl.when(s == 0)
    def _out():
        pltpu.sync_copy(acc_sh, out_hbm.at[c])


_sc_scatter = functools.partial(
    pl.kernel,
    out_type=jax.ShapeDtypeStruct((NC, N, H), jnp.float32),
    mesh=_mesh,
    scratch_types=[
        pltpu.VMEM((128,), jnp.int32),
        pltpu.VMEM((128, H), jnp.float32),
        pltpu.VMEM_SHARED((N, H), jnp.float32),
    ],
)(_sc_scatter_body)


def _stats_body(xi_ref, xj_ref, acc_ref):
    i = pl.program_id(0)
    xi = xi_ref[...]
    d = xj_ref[...] - xi
    blk = jnp.concatenate([
        jnp.sum(xi, axis=0)[None],
        jnp.sum(xi * xi, axis=0)[None],
        jnp.sum(d, axis=0)[None],
        jnp.sum(d * d, axis=0)[None],
    ], axis=0)

    @pl.when(i == 0)
    def _init():
        acc_ref[...] = blk

    @pl.when(i > 0)
    def _acc():
        acc_ref[...] += blk


def _dot(a, b):
    return lax.dot_general(a, b, (((1,), (0,)), ((), ())),
                           precision=lax.Precision.HIGHEST,
                           preferred_element_type=jnp.float32)


def _mlp1_body(xi_ref, xj_ref, coef_ref, w1_ref, y_ref, acc_ref):
    i = pl.program_id(0)
    xi = xi_ref[...]
    d = xj_ref[...] - xi
    zi = xi * coef_ref[0:1] + coef_ref[2:3]
    zi = jnp.maximum(zi, 0.2 * zi)
    zd = d * coef_ref[1:2] + coef_ref[3:4]
    zd = jnp.maximum(zd, 0.2 * zd)
    y = _dot(zi, w1_ref[0:D]) + _dot(zd, w1_ref[D:2 * D])
    y_ref[...] = y
    blk = jnp.concatenate([
        jnp.sum(y, axis=0)[None],
        jnp.sum(y * y, axis=0)[None],
    ], axis=0)

    @pl.when(i == 0)
    def _init():
        acc_ref[...] = blk

    @pl.when(i > 0)
    def _acc():
        acc_ref[...] += blk


def _mlp2_body(y_ref, wt_ref, coef2_ref, w2_ref, msg_ref):
    y = y_ref[...]
    z = y * coef2_ref[0:1] + coef2_ref[1:2]
    z = jnp.maximum(z, 0.2 * z)
    m = _dot(z, w2_ref[...])
    wt = wt_ref[...][0]
    wcol = jnp.concatenate([wt[:, q:q + 1] for q in range(QW)], axis=0)
    msg_ref[...] = m * wcol


def kernel(x, edge_index, edge_weight, g1, b1, W1, g2, b2, W2):
    src = edge_index[0].astype(jnp.int32)
    dst = edge_index[1].astype(jnp.int32)
    pad = E_PAD - E
    src2 = jnp.pad(src, (0, pad)).reshape(ROWS_PAD, 128)
    dst2 = jnp.pad(dst, (0, pad)).reshape(ROWS_PAD, 128)

    xi, xj = _sc_gather(x, src2, dst2)

    sums = pl.pallas_call(
        _stats_body,
        grid=(NBLK,),
        in_specs=[pl.BlockSpec((BE, D), lambda i: (i, 0)),
                  pl.BlockSpec((BE, D), lambda i: (i, 0))],
        out_specs=pl.BlockSpec((4, D), lambda i: (0, 0)),
        out_shape=jax.ShapeDtypeStruct((4, D), jnp.float32),
    )(xi, xj)

    eps = 1e-5
    mean1 = sums[0::2] / E
    var1 = sums[1::2] / E - mean1 * mean1
    g1m = g1.reshape(2, D)
    b1m = b1.reshape(2, D)
    a1 = g1m * lax.rsqrt(var1 + eps)
    c1 = b1m - mean1 * a1
    coef1 = jnp.concatenate([a1, c1], axis=0)

    y, ysums = pl.pallas_call(
        _mlp1_body,
        grid=(NBLK,),
        in_specs=[pl.BlockSpec((BE, D), lambda i: (i, 0)),
                  pl.BlockSpec((BE, D), lambda i: (i, 0)),
                  pl.BlockSpec((4, D), lambda i: (0, 0)),
                  pl.BlockSpec((2 * D, H), lambda i: (0, 0))],
        out_specs=[pl.BlockSpec((BE, H), lambda i: (i, 0)),
                   pl.BlockSpec((2, H), lambda i: (0, 0))],
        out_shape=[jax.ShapeDtypeStruct((E, H), jnp.float32),
                   jax.ShapeDtypeStruct((2, H), jnp.float32)],
    )(xi, xj, coef1, W1)

    mean2 = ysums[0] / E
    var2 = ysums[1] / E - mean2 * mean2
    a2 = g2 * lax.rsqrt(var2 + eps)
    c2 = b2 - mean2 * a2
    coef2 = jnp.concatenate([a2[None], c2[None]], axis=0)

    wt = edge_weight.reshape(NBLK, QW, 128).transpose(0, 2, 1)

    msg = pl.pallas_call(
        _mlp2_body,
        grid=(NBLK,),
        in_specs=[pl.BlockSpec((BE, H), lambda i: (i, 0)),
                  pl.BlockSpec((1, 128, QW), lambda i: (i, 0, 0)),
                  pl.BlockSpec((2, H), lambda i: (0, 0)),
                  pl.BlockSpec((H, H), lambda i: (0, 0))],
        out_specs=pl.BlockSpec((BE, H), lambda i: (i, 0)),
        out_shape=jax.ShapeDtypeStruct((E, H), jnp.float32),
    )(y, wt, coef2, W2)

    dsti = dst.reshape(ROWS, 128)
    zeros = jnp.zeros((N, H), jnp.float32)
    partials = _sc_scatter(msg, dsti, zeros)
    return partials[0] + partials[1]


# speedup vs baseline: 1.4381x; 1.4381x over previous
import functools

import jax
import jax.numpy as jnp
from jax import lax
from jax.experimental import pallas as pl
from jax.experimental.pallas import tpu as pltpu
from jax.experimental.pallas import tpu_sc as plsc

N = 10000
E = 320000
D = 128
H = 64

NC = 2
NS = 16
NW = NC * NS

ROWS = E // 128
ROWS_PAD = 2560
ROWS_PER_W = ROWS_PAD // NW
E_PAD = ROWS_PAD * 128
GR = 2
GROUPS = ROWS_PER_W // GR

BE = 2560
NBLK = E // BE
QW = BE // 128

def _sc_gather_body(x_hbm, src_hbm, dst_hbm, xi_hbm, xj_hbm,
                    isrc_v, idst_v, xi_v, xj_v, sem):
    c = lax.axis_index("c")
    s = lax.axis_index("s")
    wid = s * NC + c
    row0 = wid * ROWS_PER_W

    @pl.loop(0, GROUPS)
    def _group(t):
        r0 = row0 + t * GR
        pltpu.sync_copy(src_hbm.at[pl.ds(r0, GR)], isrc_v)
        pltpu.sync_copy(dst_hbm.at[pl.ds(r0, GR)], idst_v)
        cps = []
        for j in range(GR):
            cps.append(pltpu.async_copy(
                x_hbm.at[isrc_v.at[j]], xj_v.at[pl.ds(j * 128, 128)], sem))
            cps.append(pltpu.async_copy(
                x_hbm.at[idst_v.at[j]], xi_v.at[pl.ds(j * 128, 128)], sem))
        for cp in cps:
            cp.wait()
        pltpu.sync_copy(xj_v, xj_hbm.at[pl.ds(r0 * 128, GR * 128)])
        pltpu.sync_copy(xi_v, xi_hbm.at[pl.ds(r0 * 128, GR * 128)])


@functools.cache
def _sc_gather():
    mesh = plsc.VectorSubcoreMesh(core_axis_name="c", subcore_axis_name="s",
                                  num_cores=NC, num_subcores=NS)
    return pl.kernel(
        _sc_gather_body,
        out_type=[jax.ShapeDtypeStruct((E_PAD, D), jnp.float32),
                  jax.ShapeDtypeStruct((E_PAD, D), jnp.float32)],
        mesh=mesh,
        scratch_types=[
            pltpu.VMEM((GR, 128), jnp.int32),
            pltpu.VMEM((GR, 128), jnp.int32),
            pltpu.VMEM((GR * 128, D), jnp.float32),
            pltpu.VMEM((GR * 128, D), jnp.float32),
            pltpu.SemaphoreType.DMA,
        ],
    )


def _sc_scatter_body(msg_hbm, dsti_hbm, zeros_hbm, out_hbm,
                     idx_v, msg_v, acc_sh):
    c = lax.axis_index("c")
    s = lax.axis_index("s")
    wid = s * NC + c

    @pl.when(s == 0)
    def _zero():
        pltpu.sync_copy(zeros_hbm, acc_sh)

    plsc.subcore_barrier()

    niter = (ROWS + NW - 1) // NW

    @pl.loop(0, niter)
    def _it(t):
        r = wid + NW * t

        @pl.when(r < ROWS)
        def _do():
            pltpu.sync_copy(dsti_hbm.at[r], idx_v)
            pltpu.sync_copy(msg_hbm.at[pl.ds(r * 128, 128)], msg_v)
            pltpu.sync_copy(msg_v, acc_sh.at[idx_v], add=True)

    plsc.subcore_barrier()

    @pl.when(s == 0)
    def _out():
        pltpu.sync_copy(acc_sh, out_hbm.at[c])


@functools.cache
def _sc_scatter():
    mesh = plsc.VectorSubcoreMesh(core_axis_name="c", subcore_axis_name="s",
                                  num_cores=NC, num_subcores=NS)
    return pl.kernel(
        _sc_scatter_body,
        out_type=jax.ShapeDtypeStruct((NC, N, 128), jnp.float32),
        mesh=mesh,
        scratch_types=[
            pltpu.VMEM((128,), jnp.int32),
            pltpu.VMEM((128, 128), jnp.float32),
            pltpu.VMEM_SHARED((N, 128), jnp.float32),
        ],
    )


def _stats_body(xi_ref, xj_ref, acc_ref):
    i = pl.program_id(0)
    xi = xi_ref[...]
    d = xj_ref[...] - xi
    blk = jnp.concatenate([
        jnp.sum(xi, axis=0)[None],
        jnp.sum(xi * xi, axis=0)[None],
        jnp.sum(d, axis=0)[None],
        jnp.sum(d * d, axis=0)[None],
    ], axis=0)

    @pl.when(i == 0)
    def _init():
        acc_ref[...] = blk

    @pl.when(i > 0)
    def _acc():
        acc_ref[...] += blk


def _dot(a, b):
    return lax.dot_general(a, b, (((1,), (0,)), ((), ())),
                           precision=lax.Precision.HIGHEST,
                           preferred_element_type=jnp.float32)


def _mlp1_body(xi_ref, xj_ref, coef_ref, w1_ref, y_ref, acc_ref):
    i = pl.program_id(0)
    xi = xi_ref[...]
    d = xj_ref[...] - xi
    zi = xi * coef_ref[0:1] + coef_ref[2:3]
    zi = jnp.maximum(zi, 0.2 * zi)
    zd = d * coef_ref[1:2] + coef_ref[3:4]
    zd = jnp.maximum(zd, 0.2 * zd)
    y = _dot(zi, w1_ref[0:D]) + _dot(zd, w1_ref[D:2 * D])
    y_ref[...] = y
    blk = jnp.concatenate([
        jnp.sum(y, axis=0)[None],
        jnp.sum(y * y, axis=0)[None],
    ], axis=0)

    @pl.when(i == 0)
    def _init():
        acc_ref[...] = blk

    @pl.when(i > 0)
    def _acc():
        acc_ref[...] += blk


def _mlp2_body(y_ref, wt_ref, coef2_ref, w2_ref, msg_ref):
    y = y_ref[...]
    z = y * coef2_ref[0:1] + coef2_ref[1:2]
    z = jnp.maximum(z, 0.2 * z)
    m = _dot(z, w2_ref[...])
    wt = wt_ref[...][0]
    wcol = jnp.concatenate([wt[:, q:q + 1] for q in range(QW)], axis=0)
    msg_ref[...] = jnp.concatenate(
        [m * wcol, jnp.zeros((BE, 128 - H), jnp.float32)], axis=1)


def kernel(x, edge_index, edge_weight, g1, b1, W1, g2, b2, W2):
    src = edge_index[0].astype(jnp.int32)
    dst = edge_index[1].astype(jnp.int32)
    pad = E_PAD - E
    src2 = jnp.pad(src, (0, pad)).reshape(ROWS_PAD, 128)
    dst2 = jnp.pad(dst, (0, pad)).reshape(ROWS_PAD, 128)

    xi, xj = _sc_gather()(x, src2, dst2)

    sums = pl.pallas_call(
        _stats_body,
        grid=(NBLK,),
        in_specs=[pl.BlockSpec((BE, D), lambda i: (i, 0)),
                  pl.BlockSpec((BE, D), lambda i: (i, 0))],
        out_specs=pl.BlockSpec((4, D), lambda i: (0, 0)),
        out_shape=jax.ShapeDtypeStruct((4, D), jnp.float32),
    )(xi, xj)

    eps = 1e-5
    mean1 = sums[0::2] / E
    var1 = sums[1::2] / E - mean1 * mean1
    g1m = g1.reshape(2, D)
    b1m = b1.reshape(2, D)
    a1 = g1m * lax.rsqrt(var1 + eps)
    c1 = b1m - mean1 * a1
    coef1 = jnp.concatenate([a1, c1], axis=0)

    y, ysums = pl.pallas_call(
        _mlp1_body,
        grid=(NBLK,),
        in_specs=[pl.BlockSpec((BE, D), lambda i: (i, 0)),
                  pl.BlockSpec((BE, D), lambda i: (i, 0)),
                  pl.BlockSpec((4, D), lambda i: (0, 0)),
                  pl.BlockSpec((2 * D, H), lambda i: (0, 0))],
        out_specs=[pl.BlockSpec((BE, H), lambda i: (i, 0)),
                   pl.BlockSpec((2, H), lambda i: (0, 0))],
        out_shape=[jax.ShapeDtypeStruct((E, H), jnp.float32),
                   jax.ShapeDtypeStruct((2, H), jnp.float32)],
    )(xi, xj, coef1, W1)

    mean2 = ysums[0] / E
    var2 = ysums[1] / E - mean2 * mean2
    a2 = g2 * lax.rsqrt(var2 + eps)
    c2 = b2 - mean2 * a2
    coef2 = jnp.concatenate([a2[None], c2[None]], axis=0)

    wt = edge_weight.reshape(NBLK, QW, 128).transpose(0, 2, 1)

    msg = pl.pallas_call(
        _mlp2_body,
        grid=(NBLK,),
        in_specs=[pl.BlockSpec((BE, H), lambda i: (i, 0)),
                  pl.BlockSpec((1, 128, QW), lambda i: (i, 0, 0)),
                  pl.BlockSpec((2, H), lambda i: (0, 0)),
                  pl.BlockSpec((H, H), lambda i: (0, 0))],
        out_specs=pl.BlockSpec((BE, 128), lambda i: (i, 0)),
        out_shape=jax.ShapeDtypeStruct((E, 128), jnp.float32),
    )(y, wt, coef2, W2)

    dsti = dst.reshape(ROWS, 128)
    zeros = jnp.zeros((N, 128), jnp.float32)
    partials = _sc_scatter()(msg, dsti, zeros)
    return (partials[0] + partials[1])[:, :H]


# speedup vs baseline: 2.4238x; 1.6854x over previous
import functools

import jax
import jax.numpy as jnp
from jax import lax
from jax.experimental import pallas as pl
from jax.experimental.pallas import tpu as pltpu
from jax.experimental.pallas import tpu_sc as plsc

N = 10000
E = 320000
D = 128
H = 64

NC = 2
NS = 16
NW = NC * NS

ROWS = E // 128
ROWS_PAD = 2560
ROWS_PER_W = ROWS_PAD // NW
E_PAD = ROWS_PAD * 128
GR = 1
GROUPS = ROWS_PER_W // GR

BE = 2560
NBLK = E // BE
QW = BE // 128

def _sc_gather_body(x_hbm, src_hbm, dst_hbm, xi_hbm, xj_hbm,
                    isrc_v, idst_v, xi_v, xj_v, x_sh, sem):
    c = lax.axis_index("c")
    s = lax.axis_index("s")
    wid = s * NC + c
    row0 = wid * ROWS_PER_W

    @pl.when(s == 0)
    def _stage():
        pltpu.sync_copy(x_hbm, x_sh)

    plsc.subcore_barrier()

    @pl.loop(0, GROUPS)
    def _group(t):
        r0 = row0 + t * GR
        pltpu.sync_copy(src_hbm.at[pl.ds(r0, GR)], isrc_v)
        pltpu.sync_copy(dst_hbm.at[pl.ds(r0, GR)], idst_v)
        cps = []
        for j in range(GR):
            cps.append(pltpu.async_copy(
                x_sh.at[isrc_v.at[j]], xj_v.at[pl.ds(j * 128, 128)], sem))
            cps.append(pltpu.async_copy(
                x_sh.at[idst_v.at[j]], xi_v.at[pl.ds(j * 128, 128)], sem))
        for cp in cps:
            cp.wait()
        pltpu.sync_copy(xj_v, xj_hbm.at[pl.ds(r0 * 128, GR * 128)])
        pltpu.sync_copy(xi_v, xi_hbm.at[pl.ds(r0 * 128, GR * 128)])


@functools.cache
def _sc_gather():
    mesh = plsc.VectorSubcoreMesh(core_axis_name="c", subcore_axis_name="s",
                                  num_cores=NC, num_subcores=NS)
    return pl.kernel(
        _sc_gather_body,
        out_type=[jax.ShapeDtypeStruct((E_PAD, D), jnp.float32),
                  jax.ShapeDtypeStruct((E_PAD, D), jnp.float32)],
        mesh=mesh,
        scratch_types=[
            pltpu.VMEM((GR, 128), jnp.int32),
            pltpu.VMEM((GR, 128), jnp.int32),
            pltpu.VMEM((GR * 128, D), jnp.float32),
            pltpu.VMEM((GR * 128, D), jnp.float32),
            pltpu.VMEM_SHARED((N, D), jnp.float32),
            pltpu.SemaphoreType.DMA,
        ],
    )


def _sc_scatter_body(msg_hbm, dsti_hbm, zeros_hbm, out_hbm,
                     idx_v, msg_v, acc_sh):
    c = lax.axis_index("c")
    s = lax.axis_index("s")
    wid = s * NC + c

    @pl.when(s == 0)
    def _zero():
        pltpu.sync_copy(zeros_hbm, acc_sh)

    plsc.subcore_barrier()

    niter = (ROWS + NW - 1) // NW

    @pl.loop(0, niter)
    def _it(t):
        r = wid + NW * t

        @pl.when(r < ROWS)
        def _do():
            pltpu.sync_copy(dsti_hbm.at[r], idx_v)
            pltpu.sync_copy(msg_hbm.at[pl.ds(r * 128, 128)], msg_v)
            pltpu.sync_copy(msg_v, acc_sh.at[idx_v], add=True)

    plsc.subcore_barrier()

    @pl.when(s == 0)
    def _out():
        pltpu.sync_copy(acc_sh, out_hbm.at[c])


@functools.cache
def _sc_scatter():
    mesh = plsc.VectorSubcoreMesh(core_axis_name="c", subcore_axis_name="s",
                                  num_cores=NC, num_subcores=NS)
    return pl.kernel(
        _sc_scatter_body,
        out_type=jax.ShapeDtypeStruct((NC, N, 128), jnp.float32),
        mesh=mesh,
        scratch_types=[
            pltpu.VMEM((128,), jnp.int32),
            pltpu.VMEM((128, 128), jnp.float32),
            pltpu.VMEM_SHARED((N, 128), jnp.float32),
        ],
    )


def _stats_body(xi_ref, xj_ref, acc_ref):
    i = pl.program_id(0)
    xi = xi_ref[...]
    d = xj_ref[...] - xi
    blk = jnp.concatenate([
        jnp.sum(xi, axis=0)[None],
        jnp.sum(xi * xi, axis=0)[None],
        jnp.sum(d, axis=0)[None],
        jnp.sum(d * d, axis=0)[None],
    ], axis=0)

    @pl.when(i == 0)
    def _init():
        acc_ref[...] = blk

    @pl.when(i > 0)
    def _acc():
        acc_ref[...] += blk


def _dot(a, b):
    return lax.dot_general(a, b, (((1,), (0,)), ((), ())),
                           precision=lax.Precision.HIGHEST,
                           preferred_element_type=jnp.float32)


def _mlp1_body(xi_ref, xj_ref, coef_ref, w1_ref, y_ref, acc_ref):
    i = pl.program_id(0)
    xi = xi_ref[...]
    d = xj_ref[...] - xi
    zi = xi * coef_ref[0:1] + coef_ref[2:3]
    zi = jnp.maximum(zi, 0.2 * zi)
    zd = d * coef_ref[1:2] + coef_ref[3:4]
    zd = jnp.maximum(zd, 0.2 * zd)
    y = _dot(zi, w1_ref[0:D]) + _dot(zd, w1_ref[D:2 * D])
    y_ref[...] = y
    blk = jnp.concatenate([
        jnp.sum(y, axis=0)[None],
        jnp.sum(y * y, axis=0)[None],
    ], axis=0)

    @pl.when(i == 0)
    def _init():
        acc_ref[...] = blk

    @pl.when(i > 0)
    def _acc():
        acc_ref[...] += blk


def _mlp2_body(y_ref, wt_ref, coef2_ref, w2_ref, msg_ref):
    y = y_ref[...]
    z = y * coef2_ref[0:1] + coef2_ref[1:2]
    z = jnp.maximum(z, 0.2 * z)
    m = _dot(z, w2_ref[...])
    wt = wt_ref[...][0]
    wcol = jnp.concatenate([wt[:, q:q + 1] for q in range(QW)], axis=0)
    msg_ref[...] = jnp.concatenate(
        [m * wcol, jnp.zeros((BE, 128 - H), jnp.float32)], axis=1)


def kernel(x, edge_index, edge_weight, g1, b1, W1, g2, b2, W2):
    src = edge_index[0].astype(jnp.int32)
    dst = edge_index[1].astype(jnp.int32)
    pad = E_PAD - E
    src2 = jnp.pad(src, (0, pad)).reshape(ROWS_PAD, 128)
    dst2 = jnp.pad(dst, (0, pad)).reshape(ROWS_PAD, 128)

    xi, xj = _sc_gather()(x, src2, dst2)

    sums = pl.pallas_call(
        _stats_body,
        grid=(NBLK,),
        in_specs=[pl.BlockSpec((BE, D), lambda i: (i, 0)),
                  pl.BlockSpec((BE, D), lambda i: (i, 0))],
        out_specs=pl.BlockSpec((4, D), lambda i: (0, 0)),
        out_shape=jax.ShapeDtypeStruct((4, D), jnp.float32),
    )(xi, xj)

    eps = 1e-5
    mean1 = sums[0::2] / E
    var1 = sums[1::2] / E - mean1 * mean1
    g1m = g1.reshape(2, D)
    b1m = b1.reshape(2, D)
    a1 = g1m * lax.rsqrt(var1 + eps)
    c1 = b1m - mean1 * a1
    coef1 = jnp.concatenate([a1, c1], axis=0)

    y, ysums = pl.pallas_call(
        _mlp1_body,
        grid=(NBLK,),
        in_specs=[pl.BlockSpec((BE, D), lambda i: (i, 0)),
                  pl.BlockSpec((BE, D), lambda i: (i, 0)),
                  pl.BlockSpec((4, D), lambda i: (0, 0)),
                  pl.BlockSpec((2 * D, H), lambda i: (0, 0))],
        out_specs=[pl.BlockSpec((BE, H), lambda i: (i, 0)),
                   pl.BlockSpec((2, H), lambda i: (0, 0))],
        out_shape=[jax.ShapeDtypeStruct((E, H), jnp.float32),
                   jax.ShapeDtypeStruct((2, H), jnp.float32)],
    )(xi, xj, coef1, W1)

    mean2 = ysums[0] / E
    var2 = ysums[1] / E - mean2 * mean2
    a2 = g2 * lax.rsqrt(var2 + eps)
    c2 = b2 - mean2 * a2
    coef2 = jnp.concatenate([a2[None], c2[None]], axis=0)

    wt = edge_weight.reshape(NBLK, QW, 128).transpose(0, 2, 1)

    msg = pl.pallas_call(
        _mlp2_body,
        grid=(NBLK,),
        in_specs=[pl.BlockSpec((BE, H), lambda i: (i, 0)),
                  pl.BlockSpec((1, 128, QW), lambda i: (i, 0, 0)),
                  pl.BlockSpec((2, H), lambda i: (0, 0)),
                  pl.BlockSpec((H, H), lambda i: (0, 0))],
        out_specs=pl.BlockSpec((BE, 128), lambda i: (i, 0)),
        out_shape=jax.ShapeDtypeStruct((E, 128), jnp.float32),
    )(y, wt, coef2, W2)

    dsti = dst.reshape(ROWS, 128)
    zeros = jnp.zeros((N, 128), jnp.float32)
    partials = _sc_scatter()(msg, dsti, zeros)
    return (partials[0] + partials[1])[:, :H]


# speedup vs baseline: 2.8041x; 1.1569x over previous
import functools

import jax
import jax.numpy as jnp
from jax import lax
from jax.experimental import pallas as pl
from jax.experimental.pallas import tpu as pltpu
from jax.experimental.pallas import tpu_sc as plsc

N = 10000
E = 320000
D = 128
H = 64

NC = 2
NS = 16
NW = NC * NS

ROWS = E // 128
ROWS_PAD = 2560
ROWS_PER_W = ROWS_PAD // NW
E_PAD = ROWS_PAD * 128
GR = 1
GROUPS = ROWS_PER_W // GR

BE = 2560
NBLK = E // BE
QW = BE // 128

def _sc_gather_body(x_hbm, src_hbm, dst_hbm, xi_hbm, xj_hbm,
                    isrc_v, idst_v, xi_v, xj_v, x_sh, sem):
    c = lax.axis_index("c")
    s = lax.axis_index("s")
    wid = s * NC + c
    row0 = wid * ROWS_PER_W

    @pl.when(s == 0)
    def _stage():
        pltpu.sync_copy(x_hbm, x_sh)

    plsc.subcore_barrier()

    @pl.loop(0, GROUPS)
    def _group(t):
        r0 = row0 + t * GR
        pltpu.sync_copy(src_hbm.at[pl.ds(r0, GR)], isrc_v)
        pltpu.sync_copy(dst_hbm.at[pl.ds(r0, GR)], idst_v)
        cps = []
        for j in range(GR):
            cps.append(pltpu.async_copy(
                x_sh.at[isrc_v.at[j]], xj_v.at[pl.ds(j * 128, 128)], sem))
            cps.append(pltpu.async_copy(
                x_sh.at[idst_v.at[j]], xi_v.at[pl.ds(j * 128, 128)], sem))
        for cp in cps:
            cp.wait()
        pltpu.sync_copy(xj_v, xj_hbm.at[pl.ds(r0 * 128, GR * 128)])
        pltpu.sync_copy(xi_v, xi_hbm.at[pl.ds(r0 * 128, GR * 128)])


@functools.cache
def _sc_gather():
    mesh = plsc.VectorSubcoreMesh(core_axis_name="c", subcore_axis_name="s",
                                  num_cores=NC, num_subcores=NS)
    return pl.kernel(
        _sc_gather_body,
        out_type=[jax.ShapeDtypeStruct((E_PAD, D), jnp.float32),
                  jax.ShapeDtypeStruct((E_PAD, D), jnp.float32)],
        mesh=mesh,
        scratch_types=[
            pltpu.VMEM((GR, 128), jnp.int32),
            pltpu.VMEM((GR, 128), jnp.int32),
            pltpu.VMEM((GR * 128, D), jnp.float32),
            pltpu.VMEM((GR * 128, D), jnp.float32),
            pltpu.VMEM_SHARED((N, D), jnp.float32),
            pltpu.SemaphoreType.DMA,
        ],
    )


def _sc_scatter_body(msg_hbm, dsti_hbm, zeros_hbm, out_hbm,
                     idx_v, msg_v, acc_sh):
    c = lax.axis_index("c")
    s = lax.axis_index("s")
    wid = s * NC + c

    @pl.when(s == 0)
    def _zero():
        pltpu.sync_copy(zeros_hbm, acc_sh)

    plsc.subcore_barrier()

    niter = (ROWS + NW - 1) // NW

    @pl.loop(0, niter)
    def _it(t):
        r = wid + NW * t

        @pl.when(r < ROWS)
        def _do():
            pltpu.sync_copy(dsti_hbm.at[r], idx_v)
            pltpu.sync_copy(msg_hbm.at[pl.ds(r * 128, 128)], msg_v)
            pltpu.sync_copy(msg_v, acc_sh.at[idx_v], add=True)

    plsc.subcore_barrier()

    @pl.when(s == 0)
    def _out():
        pltpu.sync_copy(acc_sh, out_hbm.at[c])


@functools.cache
def _sc_scatter():
    mesh = plsc.VectorSubcoreMesh(core_axis_name="c", subcore_axis_name="s",
                                  num_cores=NC, num_subcores=NS)
    return pl.kernel(
        _sc_scatter_body,
        out_type=jax.ShapeDtypeStruct((NC, N, 128), jnp.float32),
        mesh=mesh,
        scratch_types=[
            pltpu.VMEM((128,), jnp.int32),
            pltpu.VMEM((128, 128), jnp.float32),
            pltpu.VMEM_SHARED((N, 128), jnp.float32),
        ],
    )


def _stats_body(xi_ref, xj_ref, acc_ref):
    i = pl.program_id(0)
    xi = xi_ref[...].astype(jnp.float32)
    d = xj_ref[...].astype(jnp.float32) - xi
    blk = jnp.concatenate([
        jnp.sum(xi, axis=0)[None],
        jnp.sum(xi * xi, axis=0)[None],
        jnp.sum(d, axis=0)[None],
        jnp.sum(d * d, axis=0)[None],
    ], axis=0)

    @pl.when(i == 0)
    def _init():
        acc_ref[...] = blk

    @pl.when(i > 0)
    def _acc():
        acc_ref[...] += blk


def _dot(a, b):
    return lax.dot_general(a, b, (((1,), (0,)), ((), ())),
                           precision=lax.Precision.HIGHEST,
                           preferred_element_type=jnp.float32)


def _dot16(a, b):
    return lax.dot_general(a, b, (((1,), (0,)), ((), ())),
                           preferred_element_type=jnp.float32)


def _mlp1_body(xi_ref, xj_ref, coef_ref, w1_ref, y_ref, acc_ref):
    i = pl.program_id(0)
    xi = xi_ref[...].astype(jnp.float32)
    d = xj_ref[...].astype(jnp.float32) - xi
    zi = xi * coef_ref[0:1] + coef_ref[2:3]
    zi = jnp.maximum(zi, 0.2 * zi)
    zd = d * coef_ref[1:2] + coef_ref[3:4]
    zd = jnp.maximum(zd, 0.2 * zd)
    w1 = w1_ref[...].astype(jnp.bfloat16)
    y = (_dot16(zi.astype(jnp.bfloat16), w1[0:D])
         + _dot16(zd.astype(jnp.bfloat16), w1[D:2 * D]))
    y_ref[...] = jnp.concatenate([y[:BE // 2], y[BE // 2:]], axis=1)
    blk = jnp.concatenate([
        jnp.sum(y, axis=0)[None],
        jnp.sum(y * y, axis=0)[None],
    ], axis=0)

    @pl.when(i == 0)
    def _init():
        acc_ref[...] = blk

    @pl.when(i > 0)
    def _acc():
        acc_ref[...] += blk


def _mlp2_body(y_ref, wt_ref, coef2_ref, w2_ref, msg_ref):
    y2 = y_ref[...]
    ya = y2[:, :H]
    yb = y2[:, H:]
    za = ya * coef2_ref[0:1] + coef2_ref[1:2]
    za = jnp.maximum(za, 0.2 * za)
    zb = yb * coef2_ref[0:1] + coef2_ref[1:2]
    zb = jnp.maximum(zb, 0.2 * zb)
    ma = _dot(za, w2_ref[...])
    mb = _dot(zb, w2_ref[...])
    wt = wt_ref[...][0]
    wca = jnp.concatenate([wt[:, q:q + 1] for q in range(QW // 2)], axis=0)
    wcb = jnp.concatenate([wt[:, q:q + 1] for q in range(QW // 2, QW)], axis=0)
    m = jnp.concatenate([ma * wca, mb * wcb], axis=0)
    msg_ref[...] = jnp.concatenate(
        [m, jnp.zeros((BE, 128 - H), jnp.float32)], axis=1)


def kernel(x, edge_index, edge_weight, g1, b1, W1, g2, b2, W2):
    src = edge_index[0].astype(jnp.int32)
    dst = edge_index[1].astype(jnp.int32)
    pad = E_PAD - E
    src2 = jnp.pad(src, (0, pad)).reshape(ROWS_PAD, 128)
    dst2 = jnp.pad(dst, (0, pad)).reshape(ROWS_PAD, 128)

    xi, xj = _sc_gather()(x, src2, dst2)

    sums = pl.pallas_call(
        _stats_body,
        grid=(NBLK,),
        in_specs=[pl.BlockSpec((BE, D), lambda i: (i, 0)),
                  pl.BlockSpec((BE, D), lambda i: (i, 0))],
        out_specs=pl.BlockSpec((4, D), lambda i: (0, 0)),
        out_shape=jax.ShapeDtypeStruct((4, D), jnp.float32),
    )(xi, xj)

    eps = 1e-5
    mean1 = sums[0::2] / E
    var1 = sums[1::2] / E - mean1 * mean1
    g1m = g1.reshape(2, D)
    b1m = b1.reshape(2, D)
    a1 = g1m * lax.rsqrt(var1 + eps)
    c1 = b1m - mean1 * a1
    coef1 = jnp.concatenate([a1, c1], axis=0)

    y, ysums = pl.pallas_call(
        _mlp1_body,
        grid=(NBLK,),
        in_specs=[pl.BlockSpec((BE, D), lambda i: (i, 0)),
                  pl.BlockSpec((BE, D), lambda i: (i, 0)),
                  pl.BlockSpec((4, D), lambda i: (0, 0)),
                  pl.BlockSpec((2 * D, H), lambda i: (0, 0))],
        out_specs=[pl.BlockSpec((BE // 2, 2 * H), lambda i: (i, 0)),
                   pl.BlockSpec((2, H), lambda i: (0, 0))],
        out_shape=[jax.ShapeDtypeStruct((E // 2, 2 * H), jnp.float32),
                   jax.ShapeDtypeStruct((2, H), jnp.float32)],
    )(xi, xj, coef1, W1)

    mean2 = ysums[0] / E
    var2 = ysums[1] / E - mean2 * mean2
    a2 = g2 * lax.rsqrt(var2 + eps)
    c2 = b2 - mean2 * a2
    coef2 = jnp.concatenate([a2[None], c2[None]], axis=0)

    wt = edge_weight.reshape(NBLK, QW, 128).transpose(0, 2, 1)

    msg = pl.pallas_call(
        _mlp2_body,
        grid=(NBLK,),
        in_specs=[pl.BlockSpec((BE // 2, 2 * H), lambda i: (i, 0)),
                  pl.BlockSpec((1, 128, QW), lambda i: (i, 0, 0)),
                  pl.BlockSpec((2, H), lambda i: (0, 0)),
                  pl.BlockSpec((H, H), lambda i: (0, 0))],
        out_specs=pl.BlockSpec((BE, 128), lambda i: (i, 0)),
        out_shape=jax.ShapeDtypeStruct((E, 128), jnp.float32),
    )(y, wt, coef2, W2)

    dsti = dst.reshape(ROWS, 128)
    zeros = jnp.zeros((N, 128), jnp.float32)
    partials = _sc_scatter()(msg, dsti, zeros)
    return (partials[0] + partials[1])[:, :H]
